# baseline (speedup 1.0000x reference)
import functools

import jax
import jax.numpy as jnp
from jax import lax
from jax.experimental import pallas as pl
from jax.experimental.pallas import tpu as pltpu
from jax.experimental.pallas import tpu_sc as plsc


def _argmin_body(x_ref, e_ref, idx_ref, *, K, BB):
    E = e_ref[...]
    C = x_ref.shape[1]
    T = x_ref.shape[2] * x_ref.shape[3]
    ones = jnp.ones((C, 1), jnp.float32)
    esq_col = lax.dot_general(E * E, ones, (((0,), (0,)), ((), ())),
                              precision=lax.Precision.HIGHEST,
                              preferred_element_type=jnp.float32)
    E16 = E.astype(jnp.bfloat16)
    for i in range(BB):
        xb = x_ref[i].reshape(C, T)
        simT = lax.dot_general(E16, xb.astype(jnp.bfloat16),
                               (((0,), (0,)), ((), ())),
                               preferred_element_type=jnp.float32)
        xsq_row = jnp.sum(xb * xb, axis=0, keepdims=True)
        scoresT = (xsq_row + esq_col) - 2.0 * simT
        idx_row = jnp.argmin(scoresT, axis=0).astype(jnp.int32)
        idx_ref[pl.ds(i * T, T)] = idx_row


def _sc_gather_body(idx_hbm, e_hbm, out_hbm, out2_hbm,
                    idx_v, e_v, out_v, *, HW, W, CH, NH):
    cid = lax.axis_index("c")
    sid = lax.axis_index("s")
    wid = sid * 2 + cid
    b = wid // NH
    h = wid % NH
    K = e_hbm.shape[1]
    pltpu.sync_copy(idx_hbm.at[pl.ds(b * HW, HW)], idx_v)
    for c in range(CH):
        pltpu.sync_copy(e_hbm.at[h * CH + c],
                        e_v.at[pl.ds(c * K, K)])

    UNROLL = 2

    def step(t, carry):
        for u in range(UNROLL):
            base = (t * UNROLL + u) * 16
            iv = idx_v[pl.ds(base, 16)]
            rows = [plsc.load_gather(e_v, [iv + (c * K)]) for c in range(CH)]
            for c in range(CH):
                out_v[c, pl.ds(base, 16)] = rows[c]
        return carry

    lax.fori_loop(0, HW // (16 * UNROLL), step, 0)
    pltpu.sync_copy(out_v, out_hbm.at[b, pl.ds(h * CH, CH)])
    pltpu.sync_copy(out_v, out2_hbm.at[b, pl.ds(h * CH, CH)])


def kernel(x, embeddings):
    B, C, H, W = x.shape
    Cd, K = embeddings.shape
    HW = H * W

    BB = 4
    NCHUNK = 2
    CB = B // NCHUNK

    def tc_argmin(chunk):
        return pl.pallas_call(
            functools.partial(_argmin_body, K=K, BB=BB),
            grid=(CB // BB,),
            in_specs=[
                pl.BlockSpec((BB, C, H, W),
                             lambda b, c=chunk: (c * (CB // BB) + b, 0, 0, 0)),
                pl.BlockSpec((Cd, K), lambda b: (0, 0)),
            ],
            out_specs=pl.BlockSpec((BB * HW,), lambda b: (b,)),
            out_shape=jax.ShapeDtypeStruct((CB * HW,), jnp.int32),
        )(x, embeddings)

    NH = 32 // CB
    CH = C // NH
    mesh = plsc.VectorSubcoreMesh(core_axis_name="c", subcore_axis_name="s")

    def sc_gather(idx2):
        return pl.kernel(
            functools.partial(_sc_gather_body, HW=HW, W=W, CH=CH, NH=NH),
            out_type=(jax.ShapeDtypeStruct((CB, C, HW), jnp.float32),
                      jax.ShapeDtypeStruct((CB, C, HW), jnp.float32)),
            mesh=mesh,
            compiler_params=pltpu.CompilerParams(use_tc_tiling_on_sc=False,
                                                 needs_layout_passes=False),
            scratch_types=[
                pltpu.VMEM((HW,), jnp.int32),
                pltpu.VMEM((CH * K,), jnp.float32),
                pltpu.VMEM((CH, HW), jnp.float32),
            ],
        )(idx2, embeddings)

    idxs = [tc_argmin(c) for c in range(NCHUNK)]
    scs = [sc_gather(i) for i in idxs]

    q = jnp.concatenate([s[0] for s in scs], axis=0).reshape(B, C, H, W)
    q2 = jnp.concatenate([s[1] for s in scs], axis=0).reshape(B, C, H, W)
    idx_flat = jnp.concatenate(idxs, axis=0)
    return (q, q2, idx_flat)


# speedup vs baseline: 1.0719x; 1.0719x over previous
import functools

import jax
import jax.numpy as jnp
from jax import lax
from jax.experimental import pallas as pl
from jax.experimental.pallas import tpu as pltpu
from jax.experimental.pallas import tpu_sc as plsc


def _argmin_body(x_ref, e_ref, idx_ref, *, K, BB):
    E = e_ref[...]
    C = x_ref.shape[1]
    T = x_ref.shape[2] * x_ref.shape[3]
    ones = jnp.ones((C, 1), jnp.float32)
    esq_col = lax.dot_general(E * E, ones, (((0,), (0,)), ((), ())),
                              precision=lax.Precision.HIGHEST,
                              preferred_element_type=jnp.float32)
    E16 = E.astype(jnp.bfloat16)
    for i in range(BB):
        xb = x_ref[i].reshape(C, T)
        simT = lax.dot_general(E16, xb.astype(jnp.bfloat16),
                               (((0,), (0,)), ((), ())),
                               preferred_element_type=jnp.float32)
        xsq_row = jnp.sum(xb * xb, axis=0, keepdims=True)
        scoresT = (xsq_row + esq_col) - 2.0 * simT
        idx_row = jnp.argmin(scoresT, axis=0).astype(jnp.int32)
        idx_ref[pl.ds(i * T, T)] = idx_row


def _sc_gather_body(idx_hbm, e_hbm, out_hbm, out2_hbm,
                    idx_v, e_v, out_v, *, HW, W, CH, NH):
    cid = lax.axis_index("c")
    sid = lax.axis_index("s")
    wid = sid * 2 + cid
    b = wid // NH
    h = wid % NH
    pltpu.sync_copy(idx_hbm.at[pl.ds(b * HW, HW)], idx_v)
    pltpu.sync_copy(e_hbm.at[pl.ds(h * CH, CH)], e_v)

    UNROLL = 2

    def step(t, carry):
        for u in range(UNROLL):
            base = (t * UNROLL + u) * 16
            iv = idx_v[pl.ds(base, 16)]
            rows = [plsc.load_gather(e_v, [jnp.full((16,), c, jnp.int32), iv])
                    for c in range(CH)]
            for c in range(CH):
                out_v[c, pl.ds(base, 16)] = rows[c]
        return carry

    lax.fori_loop(0, HW // (16 * UNROLL), step, 0)
    pltpu.sync_copy(out_v, out_hbm.at[b, pl.ds(h * CH, CH)])
    pltpu.sync_copy(out_v, out2_hbm.at[b, pl.ds(h * CH, CH)])


def kernel(x, embeddings):
    B, C, H, W = x.shape
    Cd, K = embeddings.shape
    HW = H * W

    BB = 4
    NCHUNK = 2
    CB = B // NCHUNK

    def tc_argmin(chunk):
        return pl.pallas_call(
            functools.partial(_argmin_body, K=K, BB=BB),
            grid=(CB // BB,),
            in_specs=[
                pl.BlockSpec((BB, C, H, W),
                             lambda b, c=chunk: (c * (CB // BB) + b, 0, 0, 0)),
                pl.BlockSpec((Cd, K), lambda b: (0, 0)),
            ],
            out_specs=pl.BlockSpec((BB * HW,), lambda b: (b,)),
            out_shape=jax.ShapeDtypeStruct((CB * HW,), jnp.int32),
        )(x, embeddings)

    NH = 32 // CB
    CH = C // NH
    mesh = plsc.VectorSubcoreMesh(core_axis_name="c", subcore_axis_name="s")

    def sc_gather(idx2):
        return pl.kernel(
            functools.partial(_sc_gather_body, HW=HW, W=W, CH=CH, NH=NH),
            out_type=(jax.ShapeDtypeStruct((CB, C, HW), jnp.float32),
                      jax.ShapeDtypeStruct((CB, C, HW), jnp.float32)),
            mesh=mesh,
            compiler_params=pltpu.CompilerParams(use_tc_tiling_on_sc=False,
                                                 needs_layout_passes=False),
            scratch_types=[
                pltpu.VMEM((HW,), jnp.int32),
                pltpu.VMEM((CH, K), jnp.float32),
                pltpu.VMEM((CH, HW), jnp.float32),
            ],
        )(idx2, embeddings)

    idxs = [tc_argmin(c) for c in range(NCHUNK)]
    scs = [sc_gather(i) for i in idxs]

    q = jnp.concatenate([s[0] for s in scs], axis=0).reshape(B, C, H, W)
    q2 = jnp.concatenate([s[1] for s in scs], axis=0).reshape(B, C, H, W)
    idx_flat = jnp.concatenate(idxs, axis=0)
    return (q, q2, idx_flat)


# speedup vs baseline: 1.0736x; 1.0016x over previous
import functools

import jax
import jax.numpy as jnp
from jax import lax
from jax.experimental import pallas as pl
from jax.experimental.pallas import tpu as pltpu
from jax.experimental.pallas import tpu_sc as plsc


def _argmin_body(x_ref, e_ref, idx_ref, *, K, BB):
    E = e_ref[...]
    C = x_ref.shape[1]
    T = x_ref.shape[2] * x_ref.shape[3]
    ones = jnp.ones((C, 1), jnp.float32)
    esq_col = lax.dot_general(E * E, ones, (((0,), (0,)), ((), ())),
                              precision=lax.Precision.HIGHEST,
                              preferred_element_type=jnp.float32)
    E16 = E.astype(jnp.bfloat16)
    for i in range(BB):
        xb = x_ref[i].reshape(C, T)
        simT = lax.dot_general(E16, xb.astype(jnp.bfloat16),
                               (((0,), (0,)), ((), ())),
                               preferred_element_type=jnp.float32)
        xsq_row = jnp.sum(xb * xb, axis=0, keepdims=True)
        scoresT = (xsq_row + esq_col) - 2.0 * simT
        idx_row = jnp.argmin(scoresT, axis=0).astype(jnp.int32)
        idx_ref[pl.ds(i * T, T)] = idx_row


def _sc_gather_body(idx_hbm, e_hbm, out_hbm, out2_hbm,
                    idx_v, e_v, out_v, *, HW, W, CH, NH):
    cid = lax.axis_index("c")
    sid = lax.axis_index("s")
    wid = sid * 2 + cid
    b = wid // NH
    h = wid % NH
    pltpu.sync_copy(idx_hbm.at[pl.ds(b * HW, HW)], idx_v)
    pltpu.sync_copy(e_hbm.at[pl.ds(h * CH, CH)], e_v)

    UNROLL = 4

    def step(t, carry):
        for u in range(UNROLL):
            base = (t * UNROLL + u) * 16
            iv = idx_v[pl.ds(base, 16)]
            rows = [plsc.load_gather(e_v, [jnp.full((16,), c, jnp.int32), iv])
                    for c in range(CH)]
            for c in range(CH):
                out_v[c, pl.ds(base, 16)] = rows[c]
        return carry

    lax.fori_loop(0, HW // (16 * UNROLL), step, 0)
    pltpu.sync_copy(out_v, out_hbm.at[b, pl.ds(h * CH, CH)])
    pltpu.sync_copy(out_v, out2_hbm.at[b, pl.ds(h * CH, CH)])


def kernel(x, embeddings):
    B, C, H, W = x.shape
    Cd, K = embeddings.shape
    HW = H * W

    BB = 4
    NCHUNK = 2
    CB = B // NCHUNK

    def tc_argmin(chunk):
        return pl.pallas_call(
            functools.partial(_argmin_body, K=K, BB=BB),
            grid=(CB // BB,),
            in_specs=[
                pl.BlockSpec((BB, C, H, W),
                             lambda b, c=chunk: (c * (CB // BB) + b, 0, 0, 0)),
                pl.BlockSpec((Cd, K), lambda b: (0, 0)),
            ],
            out_specs=pl.BlockSpec((BB * HW,), lambda b: (b,)),
            out_shape=jax.ShapeDtypeStruct((CB * HW,), jnp.int32),
        )(x, embeddings)

    NH = 32 // CB
    CH = C // NH
    mesh = plsc.VectorSubcoreMesh(core_axis_name="c", subcore_axis_name="s")

    def sc_gather(idx2):
        return pl.kernel(
            functools.partial(_sc_gather_body, HW=HW, W=W, CH=CH, NH=NH),
            out_type=(jax.ShapeDtypeStruct((CB, C, HW), jnp.float32),
                      jax.ShapeDtypeStruct((CB, C, HW), jnp.float32)),
            mesh=mesh,
            compiler_params=pltpu.CompilerParams(use_tc_tiling_on_sc=False,
                                                 needs_layout_passes=False),
            scratch_types=[
                pltpu.VMEM((HW,), jnp.int32),
                pltpu.VMEM((CH, K), jnp.float32),
                pltpu.VMEM((CH, HW), jnp.float32),
            ],
        )(idx2, embeddings)

    idxs = [tc_argmin(c) for c in range(NCHUNK)]
    scs = [sc_gather(i) for i in idxs]

    q = jnp.concatenate([s[0] for s in scs], axis=0).reshape(B, C, H, W)
    q2 = jnp.concatenate([s[1] for s in scs], axis=0).reshape(B, C, H, W)
    idx_flat = jnp.concatenate(idxs, axis=0)
    return (q, q2, idx_flat)
